# Initial kernel scaffold; baseline (speedup 1.0000x reference)
import jax
import jax.numpy as jnp
from jax.experimental import pallas as pl


def kernel(student_feats, teacher_feats, mask, original_x, center):
    raise NotImplementedError("write your pallas kernel here")


# speedup vs baseline: 1.0643x; 1.0643x over previous
import jax
import jax.numpy as jnp
from jax.experimental import pallas as pl


BH = 16


def _loss_kernel(s_ref, t_ref, m_ref, ox_ref, c_ref, sum_ref, cnt_ref):
    b = pl.program_id(0)
    h = pl.program_id(1)

    @pl.when(jnp.logical_and(b == 0, h == 0))
    def _init():
        sum_ref[...] = jnp.zeros((1, 1), jnp.float32)
        cnt_ref[...] = jnp.zeros((1, 1), jnp.float32)

    s = s_ref[0]
    t = t_ref[0] - c_ref[...]
    dot = jnp.sum(s * t, axis=0)
    ns2 = jnp.sum(s * s, axis=0)
    nt2 = jnp.sum(t * t, axis=0)
    eps = 1e-8
    denom = jnp.maximum(jnp.sqrt(ns2), eps) * jnp.maximum(jnp.sqrt(nt2), eps)
    loss_px = 1.0 - dot / denom

    validf = ox_ref[0, 0] * m_ref[0]
    sum_ref[...] += jnp.sum(loss_px * validf).reshape(1, 1)
    cnt_ref[...] += jnp.sum(validf).reshape(1, 1)


def kernel(student_feats, teacher_feats, mask, original_x, center):
    B, D, H, W = student_feats.shape
    active = (original_x != 0).astype(jnp.float32)
    not_mask = jnp.logical_not(mask).astype(jnp.float32)
    center3 = center.reshape(D, 1, 1)

    grid = (B, H // BH)
    out_spec = pl.BlockSpec((1, 1), lambda b, h: (0, 0))
    loss_sum, cnt = pl.pallas_call(
        _loss_kernel,
        grid=grid,
        in_specs=[
            pl.BlockSpec((1, D, BH, W), lambda b, h: (b, 0, h, 0)),
            pl.BlockSpec((1, D, BH, W), lambda b, h: (b, 0, h, 0)),
            pl.BlockSpec((1, BH, W), lambda b, h: (b, h, 0)),
            pl.BlockSpec((1, 1, BH, W), lambda b, h: (b, 0, h, 0)),
            pl.BlockSpec((D, 1, 1), lambda b, h: (0, 0, 0)),
        ],
        out_specs=[out_spec, out_spec],
        out_shape=[
            jax.ShapeDtypeStruct((1, 1), jnp.float32),
            jax.ShapeDtypeStruct((1, 1), jnp.float32),
        ],
    )(student_feats, teacher_feats, not_mask, active, center3)

    s = loss_sum[0, 0]
    c = cnt[0, 0]
    return jnp.where(c > 0, s / jnp.maximum(c, 1.0), jnp.float32(0.0))
